# baseline (speedup 1.0000x reference)
import functools

import jax
import jax.numpy as jnp
from jax import lax
from jax.experimental import pallas as pl
from jax.experimental.pallas import tpu as pltpu
from jax.experimental.pallas import tpu_sc as plsc

_G = 4


def kernel(x, weight):
    B, H = x.shape
    V, D = weight.shape
    HP = 56
    info = plsc.get_sparse_core_info()
    nw = info.num_cores * info.num_subcores
    rows_w = B // nw
    ng = rows_w // _G
    assert B == nw * ng * _G and ng % 2 == 0, (B, nw, ng)

    xp = jnp.pad(x, ((0, 0), (0, 64 - H))).astype(jnp.int32)
    mesh = plsc.VectorSubcoreMesh(core_axis_name="c", subcore_axis_name="s")

    @functools.partial(
        pl.kernel,
        mesh=mesh,
        out_type=jax.ShapeDtypeStruct((B, HP, 128), jnp.float32),
        scratch_types=[
            pltpu.VMEM((rows_w, 64), jnp.int32),
            pltpu.VMEM((2, _G * 64, 128), jnp.float32),
            pltpu.SemaphoreType.DMA,
            pltpu.SemaphoreType.DMA,
            pltpu.SemaphoreType.DMA,
            pltpu.SemaphoreType.DMA,
        ],
        compiler_params=pltpu.CompilerParams(use_tc_tiling_on_sc=False),
    )
    def run(xp_hbm, w_hbm, out_hbm, idx_v, rows_v,
            gsem0, gsem1, osem0, osem1):
        wid = lax.axis_index("s") * info.num_cores + lax.axis_index("c")
        base = wid * rows_w
        gsems = (gsem0, gsem1)
        osems = (osem0, osem1)
        pltpu.sync_copy(xp_hbm.at[pl.ds(wid * rows_w, rows_w)], idx_v)

        def issue_gathers(g, bank):
            for j in range(_G):
                r = g * _G + j
                pltpu.async_copy(
                    w_hbm.at[idx_v.at[r]],
                    rows_v.at[bank, pl.ds(j * 64, 64)],
                    gsems[bank],
                )

        def wait_gathers(bank):
            pltpu.make_async_copy(
                w_hbm.at[pl.ds(0, _G * 64)],
                rows_v.at[bank],
                gsems[bank],
            ).wait()

        def issue_out(g, bank):
            for j in range(_G):
                pltpu.async_copy(
                    rows_v.at[bank, pl.ds(j * 64, HP)],
                    out_hbm.at[base + g * _G + j],
                    osems[bank],
                )

        def wait_out(bank):
            pltpu.make_async_copy(
                w_hbm.at[pl.ds(0, _G * HP)],
                rows_v.at[bank, pl.ds(0, _G * HP)],
                osems[bank],
            ).wait()

        issue_gathers(0, 0)
        issue_gathers(1, 1)
        wait_gathers(0)
        issue_out(0, 0)

        def pair(p, carry):
            for b, g in ((1, 2 * p + 1), (0, 2 * p + 2)):
                wait_out(1 - b)
                issue_gathers(g + 1, 1 - b)
                wait_gathers(b)
                issue_out(g, b)
            return carry

        lax.fori_loop(0, (ng - 2) // 2, pair, 0)

        wait_out(0)
        wait_gathers(1)
        issue_out(ng - 1, 1)
        wait_out(1)

    wp = jnp.pad(weight, ((0, 0), (0, 128 - D)))
    out3 = run(xp, wp)
    return out3[:, :H, :D]


# speedup vs baseline: 9.6254x; 9.6254x over previous
import functools

import jax
import jax.numpy as jnp
from jax import lax
from jax.experimental import pallas as pl
from jax.experimental.pallas import tpu as pltpu
from jax.experimental.pallas import tpu_sc as plsc

_G = 4


def kernel(x, weight):
    B, H = x.shape
    V, D = weight.shape
    HP = 56
    info = plsc.get_sparse_core_info()
    nw = info.num_cores * info.num_subcores
    rows_w = B // nw
    ng = rows_w // _G
    assert B == nw * ng * _G and ng % 2 == 0, (B, nw, ng)

    xp = jnp.concatenate([x, x[:, : 64 - H]], axis=1).astype(jnp.int32)
    mesh = plsc.VectorSubcoreMesh(core_axis_name="c", subcore_axis_name="s")

    @functools.partial(
        pl.kernel,
        mesh=mesh,
        out_type=jax.ShapeDtypeStruct((B, HP, 128), jnp.float32),
        scratch_types=[
            pltpu.VMEM((rows_w, 64), jnp.int32),
            pltpu.VMEM((2, _G * 64, 128), jnp.float32),
            pltpu.SemaphoreType.DMA,
            pltpu.SemaphoreType.DMA,
            pltpu.SemaphoreType.DMA,
            pltpu.SemaphoreType.DMA,
        ],
        compiler_params=pltpu.CompilerParams(use_tc_tiling_on_sc=False),
    )
    def run(xp_hbm, w_hbm, out_hbm, idx_v, rows_v,
            gsem0, gsem1, osem0, osem1):
        wid = lax.axis_index("s") * info.num_cores + lax.axis_index("c")
        base = wid * rows_w
        gsems = (gsem0, gsem1)
        osems = (osem0, osem1)
        pltpu.sync_copy(xp_hbm.at[pl.ds(wid * rows_w, rows_w)], idx_v)

        def issue_gathers(g, bank):
            for j in range(_G):
                r = g * _G + j
                pltpu.async_copy(
                    w_hbm.at[idx_v.at[r]],
                    rows_v.at[bank, pl.ds(j * 64, 64)],
                    gsems[bank],
                )

        def wait_gathers(bank):
            pltpu.make_async_copy(
                w_hbm.at[pl.ds(0, _G * 64)],
                rows_v.at[bank],
                gsems[bank],
            ).wait()

        def issue_out(g, bank):
            for j in range(_G):
                pltpu.async_copy(
                    rows_v.at[bank, pl.ds(j * 64, HP)],
                    out_hbm.at[base + g * _G + j],
                    osems[bank],
                )

        def wait_out(bank):
            pltpu.make_async_copy(
                w_hbm.at[pl.ds(0, _G * HP)],
                rows_v.at[bank, pl.ds(0, _G * HP)],
                osems[bank],
            ).wait()

        issue_gathers(0, 0)
        issue_gathers(1, 1)
        wait_gathers(0)
        issue_out(0, 0)

        def pair(p, carry):
            for b, g in ((1, 2 * p + 1), (0, 2 * p + 2)):
                wait_out(1 - b)
                issue_gathers(g + 1, 1 - b)
                wait_gathers(b)
                issue_out(g, b)
            return carry

        lax.fori_loop(0, (ng - 2) // 2, pair, 0)

        wait_out(0)
        wait_gathers(1)
        issue_out(ng - 1, 1)
        wait_out(1)

    wp = jnp.pad(weight, ((0, 0), (0, 128 - D)))
    out3 = run(xp, wp)
    return out3[:, :H, :D]
